# initial kernel scaffold (unmeasured)
import functools

import jax
import jax.numpy as jnp
from jax import lax
from jax.experimental import pallas as pl
from jax.experimental.pallas import tpu as pltpu

NZ = 4
PAD = 1152


def _a2a_body(blocks_ref, counts_ref, rb_ref, rcnt_ref,
              bsend, brecv, csend, crecv):
    my_x = lax.axis_index("x")
    my_y = lax.axis_index("y")
    my_z = lax.axis_index("z")

    barrier_sem = pltpu.get_barrier_semaphore()
    for delta in (1, 2, 3):
        zt = (my_z + delta) % NZ
        pl.semaphore_signal(
            barrier_sem, inc=1,
            device_id=(my_x, my_y, zt),
            device_id_type=pl.DeviceIdType.MESH,
        )
    pl.semaphore_wait(barrier_sem, NZ - 1)

    rb_ref[0] = blocks_ref[0]
    rcnt_ref[0] = counts_ref[0]

    rdmas = []
    for delta in (1, 2, 3):
        zt = (my_z + delta) % NZ
        blk = pltpu.make_async_remote_copy(
            src_ref=blocks_ref.at[delta],
            dst_ref=rb_ref.at[delta],
            send_sem=bsend.at[delta],
            recv_sem=brecv.at[delta],
            device_id=(my_x, my_y, zt),
            device_id_type=pl.DeviceIdType.MESH,
        )
        cnt = pltpu.make_async_remote_copy(
            src_ref=counts_ref.at[delta],
            dst_ref=rcnt_ref.at[delta],
            send_sem=csend.at[delta],
            recv_sem=crecv.at[delta],
            device_id=(my_x, my_y, zt),
            device_id_type=pl.DeviceIdType.MESH,
        )
        blk.start()
        cnt.start()
        rdmas.append(blk)
        rdmas.append(cnt)

    for r in rdmas:
        r.wait()


def kernel(x, dest):
    m, n = x.shape
    my_z = lax.axis_index("z")

    xb = x.astype(jnp.bfloat16)
    order = jnp.argsort(dest, stable=True)
    xs = xb[order]

    counts = jnp.sum(dest[None, :] == jnp.arange(NZ)[:, None], axis=1)
    starts = jnp.concatenate(
        [jnp.zeros((1,), jnp.int32), jnp.cumsum(counts)[:-1].astype(jnp.int32)]
    )
    xs_pad = jnp.concatenate([xs, jnp.zeros((PAD, n), jnp.bfloat16)], axis=0)
    blocks_by_dst = jnp.stack(
        [lax.dynamic_slice(xs_pad, (starts[d], 0), (PAD, n)) for d in range(NZ)]
    )

    delta_to_dst = (my_z + jnp.arange(NZ)) % NZ
    blocks = jnp.take(blocks_by_dst, delta_to_dst, axis=0)
    counts_tile = jnp.broadcast_to(
        jnp.take(counts, delta_to_dst)[:, None, None].astype(jnp.int32),
        (NZ, 8, 128),
    )

    rb, rcnt = pl.pallas_call(
        _a2a_body,
        out_shape=[
            jax.ShapeDtypeStruct((NZ, PAD, n), jnp.bfloat16),
            jax.ShapeDtypeStruct((NZ, 8, 128), jnp.int32),
        ],
        in_specs=[
            pl.BlockSpec(memory_space=pltpu.VMEM),
            pl.BlockSpec(memory_space=pltpu.VMEM),
        ],
        out_specs=[
            pl.BlockSpec(memory_space=pltpu.VMEM),
            pl.BlockSpec(memory_space=pltpu.VMEM),
        ],
        scratch_shapes=[
            pltpu.SemaphoreType.DMA((NZ,)),
            pltpu.SemaphoreType.DMA((NZ,)),
            pltpu.SemaphoreType.DMA((NZ,)),
            pltpu.SemaphoreType.DMA((NZ,)),
        ],
        compiler_params=pltpu.CompilerParams(collective_id=0),
    )(blocks, counts_tile)

    slot_of_src = (my_z - jnp.arange(NZ)) % NZ
    c_src = jnp.take(rcnt[:, 0, 0], slot_of_src)
    off = jnp.concatenate(
        [jnp.zeros((1,), jnp.int32), jnp.cumsum(c_src)[:-1].astype(jnp.int32)]
    )
    i = jnp.arange(m)
    s_i = jnp.sum(i[:, None] >= off[None, :], axis=1) - 1
    row_in_blk = i - jnp.take(off, s_i)
    flat = jnp.take(slot_of_src, s_i) * PAD + row_in_blk
    out = jnp.take(rb.reshape(NZ * PAD, n), flat, axis=0)
    return out.astype(jnp.float32)


# baseline (device time: 157732 ns/iter reference)
import functools

import jax
import jax.numpy as jnp
from jax import lax
from jax.experimental import pallas as pl
from jax.experimental.pallas import tpu as pltpu

NZ = 4
PAD = 1152


def _a2a_body(blocks_ref, counts_ref, rb_ref, rcnt_ref,
              bsend, brecv, csend, crecv):
    my_x = lax.axis_index("x")
    my_y = lax.axis_index("y")
    my_z = lax.axis_index("z")

    barrier_sem = pltpu.get_barrier_semaphore()
    for delta in (1, 2, 3):
        zt = (my_z + delta) % NZ
        pl.semaphore_signal(
            barrier_sem, inc=1,
            device_id=(my_x, my_y, zt),
            device_id_type=pl.DeviceIdType.MESH,
        )
    pl.semaphore_wait(barrier_sem, NZ - 1)

    rb_ref[0] = blocks_ref[0]
    rcnt_ref[0] = counts_ref[0]

    rdmas = []
    for delta in (1, 2, 3):
        zt = (my_z + delta) % NZ
        blk = pltpu.make_async_remote_copy(
            src_ref=blocks_ref.at[delta],
            dst_ref=rb_ref.at[delta],
            send_sem=bsend.at[delta],
            recv_sem=brecv.at[delta],
            device_id=(my_x, my_y, zt),
            device_id_type=pl.DeviceIdType.MESH,
        )
        cnt = pltpu.make_async_remote_copy(
            src_ref=counts_ref.at[delta],
            dst_ref=rcnt_ref.at[delta],
            send_sem=csend.at[delta],
            recv_sem=crecv.at[delta],
            device_id=(my_x, my_y, zt),
            device_id_type=pl.DeviceIdType.MESH,
        )
        blk.start()
        cnt.start()
        rdmas.append(blk)
        rdmas.append(cnt)

    for r in rdmas:
        r.wait()


def kernel(x, dest):
    m, n = x.shape
    my_z = lax.axis_index("z")

    xb = x.astype(jnp.bfloat16)
    order = jnp.argsort(dest, stable=True).astype(jnp.int32)
    counts = jnp.sum(
        dest[None, :] == jnp.arange(NZ, dtype=dest.dtype)[:, None], axis=1
    ).astype(jnp.int32)
    starts = jnp.concatenate(
        [jnp.zeros((1,), jnp.int32), jnp.cumsum(counts)[:-1].astype(jnp.int32)]
    )
    delta_to_dst = (my_z + jnp.arange(NZ)) % NZ
    gidx = jnp.take(starts, delta_to_dst)[:, None] + jnp.arange(PAD)[None, :]
    row_ids = jnp.take(order, jnp.minimum(gidx, m - 1))
    blocks = jnp.take(xb, row_ids.reshape(-1), axis=0).reshape(NZ, PAD, n)
    counts_tile = jnp.broadcast_to(
        jnp.take(counts, delta_to_dst)[:, None, None].astype(jnp.int32),
        (NZ, 8, 128),
    )

    rb, rcnt = pl.pallas_call(
        _a2a_body,
        out_shape=[
            jax.ShapeDtypeStruct((NZ, PAD, n), jnp.bfloat16),
            jax.ShapeDtypeStruct((NZ, 8, 128), jnp.int32),
        ],
        in_specs=[
            pl.BlockSpec(memory_space=pltpu.VMEM),
            pl.BlockSpec(memory_space=pltpu.VMEM),
        ],
        out_specs=[
            pl.BlockSpec(memory_space=pltpu.VMEM),
            pl.BlockSpec(memory_space=pltpu.VMEM),
        ],
        scratch_shapes=[
            pltpu.SemaphoreType.DMA((NZ,)),
            pltpu.SemaphoreType.DMA((NZ,)),
            pltpu.SemaphoreType.DMA((NZ,)),
            pltpu.SemaphoreType.DMA((NZ,)),
        ],
        compiler_params=pltpu.CompilerParams(collective_id=0),
    )(blocks, counts_tile)

    slot_of_src = (my_z - jnp.arange(NZ)) % NZ
    c_src = jnp.take(rcnt[:, 0, 0], slot_of_src)
    off = jnp.concatenate(
        [jnp.zeros((1,), jnp.int32), jnp.cumsum(c_src)[:-1].astype(jnp.int32)]
    )
    i = jnp.arange(m)
    s_i = jnp.sum(i[:, None] >= off[None, :], axis=1) - 1
    row_in_blk = i - jnp.take(off, s_i)
    flat = jnp.take(slot_of_src, s_i) * PAD + row_in_blk
    return jnp.take(rb.reshape(NZ * PAD, n), flat, axis=0)


# device time: 115187 ns/iter; 1.3694x vs baseline; 1.3694x over previous
import jax
import jax.numpy as jnp
from jax import lax
from jax.experimental import pallas as pl
from jax.experimental.pallas import tpu as pltpu

NZ = 4
PAD = 1152
KT = 512
W = PAD + 64


def _body(x_ref, gpos_ref, counts_ref, out_ref,
          blocks_ref, rb_ref, rcnt_ref,
          bsend, brecv, csend, crecv):
    m, n = x_ref.shape
    my_x = lax.axis_index("x")
    my_y = lax.axis_index("y")
    my_z = lax.axis_index("z")

    barrier_sem = pltpu.get_barrier_semaphore()
    for delta in (1, 2, 3):
        zt = (my_z + delta) % NZ
        pl.semaphore_signal(
            barrier_sem, inc=1,
            device_id=(my_x, my_y, zt),
            device_id_type=pl.DeviceIdType.MESH,
        )
    pl.semaphore_wait(barrier_sem, NZ - 1)

    def make_rdma(delta):
        zt = (my_z + delta) % NZ
        blk = pltpu.make_async_remote_copy(
            src_ref=blocks_ref.at[delta],
            dst_ref=rb_ref.at[delta],
            send_sem=bsend.at[delta],
            recv_sem=brecv.at[delta],
            device_id=(my_x, my_y, zt),
            device_id_type=pl.DeviceIdType.MESH,
        )
        cnt = pltpu.make_async_remote_copy(
            src_ref=counts_ref.at[delta],
            dst_ref=rcnt_ref.at[delta],
            send_sem=csend.at[delta],
            recv_sem=crecv.at[delta],
            device_id=(my_x, my_y, zt),
            device_id_type=pl.DeviceIdType.MESH,
        )
        return blk, cnt

    NH = n // 2

    def build_block(delta):
        acc = [None, None]
        for k in range(0, m, KT):
            gk = gpos_ref[:, pl.ds(k, KT)]
            iota = lax.broadcasted_iota(jnp.int32, (PAD, KT), 0)
            oh = (iota + (delta * PAD) == gk).astype(jnp.bfloat16)
            for h in range(2):
                xk = x_ref[pl.ds(k, KT), pl.ds(h * NH, NH)].astype(
                    jnp.bfloat16
                )
                part = jax.lax.dot_general(
                    oh, xk,
                    dimension_numbers=(((1,), (0,)), ((), ())),
                    preferred_element_type=jnp.float32,
                ).astype(jnp.bfloat16)
                acc[h] = part if acc[h] is None else acc[h] + part
        for h in range(2):
            blocks_ref[delta, :, pl.ds(h * NH, NH)] = acc[h]

    rdmas = []
    for delta in (1, 2, 3):
        build_block(delta)
        blk, cnt = make_rdma(delta)
        blk.start()
        cnt.start()
        rdmas.append(blk)
        rdmas.append(cnt)
    build_block(0)
    rb_ref[0] = blocks_ref[0]
    rcnt_ref[0] = counts_ref[0]

    zchunk = jnp.zeros((KT, n), jnp.bfloat16)
    for k in range(0, m, KT):
        out_ref[pl.ds(k, KT), :] = zchunk

    for r in rdmas:
        r.wait()

    c_slot = [rcnt_ref[d, 0, 0] for d in range(NZ)]
    off = jnp.int32(0)
    off_slot = [jnp.int32(0)] * NZ
    for s in range(NZ):
        slot = (my_z - s) % NZ
        c_s = jnp.int32(0)
        for d in range(NZ):
            c_s = jnp.where(slot == d, c_slot[d], c_s)
            off_slot[d] = jnp.where(slot == d, off, off_slot[d])
        off = off + c_s

    for delta in range(NZ):
        o = off_slot[delta]
        o8 = (o // 8) * 8
        w = pl.multiple_of(jnp.minimum(o8, m - W), 8)
        for h in range(2):
            padded = jnp.concatenate(
                [
                    rb_ref[delta, :, pl.ds(h * NH, NH)],
                    jnp.zeros((W - PAD, NH), jnp.bfloat16),
                ],
                axis=0,
            )
            rolled = pltpu.roll(padded, o - w, axis=0)
            cur = out_ref[pl.ds(w, W), pl.ds(h * NH, NH)]
            out_ref[pl.ds(w, W), pl.ds(h * NH, NH)] = cur + rolled


def kernel(x, dest):
    m, n = x.shape
    my_z = lax.axis_index("z")

    oneh = (dest[:, None] == jnp.arange(NZ, dtype=dest.dtype)[None, :])
    pos = jnp.sum((jnp.cumsum(oneh, axis=0) - 1) * oneh, axis=1)
    delta_of_row = (dest - my_z) % NZ
    gpos = (delta_of_row * PAD + pos).astype(jnp.int32)[None, :]

    counts = jnp.sum(oneh, axis=0)
    perm = (jnp.arange(NZ)[:, None] == (my_z + jnp.arange(NZ)[None, :]) % NZ)
    counts_delta = jnp.sum(counts[:, None] * perm, axis=0)
    counts_tile = jnp.broadcast_to(
        counts_delta[:, None, None].astype(jnp.int32), (NZ, 8, 128)
    )

    return pl.pallas_call(
        _body,
        out_shape=jax.ShapeDtypeStruct((m, n), jnp.bfloat16),
        in_specs=[
            pl.BlockSpec(memory_space=pltpu.VMEM),
            pl.BlockSpec(memory_space=pltpu.VMEM),
            pl.BlockSpec(memory_space=pltpu.VMEM),
        ],
        out_specs=pl.BlockSpec(memory_space=pltpu.VMEM),
        scratch_shapes=[
            pltpu.VMEM((NZ, PAD, n), jnp.bfloat16),
            pltpu.VMEM((NZ, PAD, n), jnp.bfloat16),
            pltpu.VMEM((NZ, 8, 128), jnp.int32),
            pltpu.SemaphoreType.DMA((NZ,)),
            pltpu.SemaphoreType.DMA((NZ,)),
            pltpu.SemaphoreType.DMA((NZ,)),
            pltpu.SemaphoreType.DMA((NZ,)),
        ],
        compiler_params=pltpu.CompilerParams(
            collective_id=0,
            vmem_limit_bytes=25 * 1024 * 1024,
        ),
    )(x, gpos, counts_tile)


# device time: 109783 ns/iter; 1.4368x vs baseline; 1.0492x over previous
import jax
import jax.numpy as jnp
from jax import lax
from jax.experimental import pallas as pl
from jax.experimental.pallas import tpu as pltpu

NZ = 4
PAD = 1088
KT = 512
W = PAD + 64


def _body(x_ref, gpos_ref, counts_ref, out_ref,
          blocks_ref, rb_ref, rcnt_ref,
          bsend, brecv, csend, crecv):
    m, n = x_ref.shape
    my_x = lax.axis_index("x")
    my_y = lax.axis_index("y")
    my_z = lax.axis_index("z")

    barrier_sem = pltpu.get_barrier_semaphore()
    for delta in (1, 2, 3):
        zt = (my_z + delta) % NZ
        pl.semaphore_signal(
            barrier_sem, inc=1,
            device_id=(my_x, my_y, zt),
            device_id_type=pl.DeviceIdType.MESH,
        )
    pl.semaphore_wait(barrier_sem, NZ - 1)

    def make_rdma(delta):
        zt = (my_z + delta) % NZ
        blk = pltpu.make_async_remote_copy(
            src_ref=blocks_ref.at[delta],
            dst_ref=rb_ref.at[delta],
            send_sem=bsend.at[delta],
            recv_sem=brecv.at[delta],
            device_id=(my_x, my_y, zt),
            device_id_type=pl.DeviceIdType.MESH,
        )
        cnt = pltpu.make_async_remote_copy(
            src_ref=counts_ref.at[delta],
            dst_ref=rcnt_ref.at[delta],
            send_sem=csend.at[delta],
            recv_sem=crecv.at[delta],
            device_id=(my_x, my_y, zt),
            device_id_type=pl.DeviceIdType.MESH,
        )
        return blk, cnt

    NH = n // 2

    def build_block(delta):
        acc = [None, None]
        for k in range(0, m, KT):
            gk = gpos_ref[:, pl.ds(k, KT)]
            iota = lax.broadcasted_iota(jnp.int32, (PAD, KT), 0)
            oh = (iota + (delta * PAD) == gk).astype(jnp.bfloat16)
            for h in range(2):
                xk = x_ref[pl.ds(k, KT), pl.ds(h * NH, NH)].astype(
                    jnp.bfloat16
                )
                part = jax.lax.dot_general(
                    oh, xk,
                    dimension_numbers=(((1,), (0,)), ((), ())),
                    preferred_element_type=jnp.float32,
                ).astype(jnp.bfloat16)
                acc[h] = part if acc[h] is None else acc[h] + part
        for h in range(2):
            blocks_ref[delta, :, pl.ds(h * NH, NH)] = acc[h]

    rdmas = []
    for delta in (1, 2, 3):
        build_block(delta)
        blk, cnt = make_rdma(delta)
        blk.start()
        cnt.start()
        rdmas.append(blk)
        rdmas.append(cnt)
    build_block(0)
    rb_ref[0] = blocks_ref[0]
    rcnt_ref[0] = counts_ref[0]

    zchunk = jnp.zeros((KT, n), jnp.bfloat16)
    for k in range(0, m, KT):
        out_ref[pl.ds(k, KT), :] = zchunk

    for r in rdmas:
        r.wait()

    c_slot = [rcnt_ref[d, 0, 0] for d in range(NZ)]
    off = jnp.int32(0)
    off_slot = [jnp.int32(0)] * NZ
    for s in range(NZ):
        slot = (my_z - s) % NZ
        c_s = jnp.int32(0)
        for d in range(NZ):
            c_s = jnp.where(slot == d, c_slot[d], c_s)
            off_slot[d] = jnp.where(slot == d, off, off_slot[d])
        off = off + c_s

    for delta in range(NZ):
        o = off_slot[delta]
        o8 = (o // 8) * 8
        w = pl.multiple_of(jnp.minimum(o8, m - W), 8)
        for h in range(2):
            padded = jnp.concatenate(
                [
                    rb_ref[delta, :, pl.ds(h * NH, NH)],
                    jnp.zeros((W - PAD, NH), jnp.bfloat16),
                ],
                axis=0,
            )
            rolled = pltpu.roll(padded, o - w, axis=0)
            cur = out_ref[pl.ds(w, W), pl.ds(h * NH, NH)]
            out_ref[pl.ds(w, W), pl.ds(h * NH, NH)] = cur + rolled


def kernel(x, dest):
    m, n = x.shape
    my_z = lax.axis_index("z")

    oneh = (dest[:, None] == jnp.arange(NZ, dtype=dest.dtype)[None, :])
    pos = jnp.sum((jnp.cumsum(oneh, axis=0) - 1) * oneh, axis=1)
    delta_of_row = (dest - my_z) % NZ
    gpos = (delta_of_row * PAD + pos).astype(jnp.int32)[None, :]

    counts = jnp.sum(oneh, axis=0)
    perm = (jnp.arange(NZ)[:, None] == (my_z + jnp.arange(NZ)[None, :]) % NZ)
    counts_delta = jnp.sum(counts[:, None] * perm, axis=0)
    counts_tile = jnp.broadcast_to(
        counts_delta[:, None, None].astype(jnp.int32), (NZ, 8, 128)
    )

    return pl.pallas_call(
        _body,
        out_shape=jax.ShapeDtypeStruct((m, n), jnp.bfloat16),
        in_specs=[
            pl.BlockSpec(memory_space=pltpu.VMEM),
            pl.BlockSpec(memory_space=pltpu.VMEM),
            pl.BlockSpec(memory_space=pltpu.VMEM),
        ],
        out_specs=pl.BlockSpec(memory_space=pltpu.VMEM),
        scratch_shapes=[
            pltpu.VMEM((NZ, PAD, n), jnp.bfloat16),
            pltpu.VMEM((NZ, PAD, n), jnp.bfloat16),
            pltpu.VMEM((NZ, 8, 128), jnp.int32),
            pltpu.SemaphoreType.DMA((NZ,)),
            pltpu.SemaphoreType.DMA((NZ,)),
            pltpu.SemaphoreType.DMA((NZ,)),
            pltpu.SemaphoreType.DMA((NZ,)),
        ],
        compiler_params=pltpu.CompilerParams(
            collective_id=0,
            vmem_limit_bytes=25 * 1024 * 1024,
        ),
    )(x, gpos, counts_tile)
